# baseline (speedup 1.0000x reference)
import functools

import jax
import jax.numpy as jnp
from jax import lax
from jax.experimental import pallas as pl
from jax.experimental.pallas import tpu as pltpu
from jax.experimental.pallas import tpu_sc as plsc

N_NODES = 100000
DIM = 128
B_POS = 16384
B_NEG = 81920
EPS = 1e-7

E = B_POS + B_NEG
TOT = 2 * E
NW = 32
PER_W = TOT // NW
CH = 128
N_CH = PER_W // CH

@functools.cache
def _make_sc_gather():
    mesh = plsc.VectorSubcoreMesh(core_axis_name="c", subcore_axis_name="s")

    @functools.partial(
        pl.kernel,
        mesh=mesh,
        out_type=(
            jax.ShapeDtypeStruct((TOT, DIM), jnp.float32),
            jax.ShapeDtypeStruct((TOT,), jnp.float32),
        ),
        scratch_types=[
            pltpu.VMEM((PER_W,), jnp.int32),
            pltpu.VMEM((CH, DIM), jnp.float32),
            pltpu.VMEM((CH,), jnp.float32),
            pltpu.SemaphoreType.DMA,
            pltpu.SemaphoreType.DMA,
        ],
    )
    def _sc_gather(embs_hbm, offs_hbm, idx_hbm, rows_out, offs_out,
                   idx_v, rowbuf, offbuf, sem_r, sem_o):
        wid = lax.axis_index("s") * 2 + lax.axis_index("c")
        base = wid * PER_W
        pltpu.sync_copy(idx_hbm.at[pl.ds(base, PER_W)], idx_v)

        def body(j, carry):
            off = j * CH
            idx_c = idx_v.at[pl.ds(off, CH)]
            cr = pltpu.async_copy(embs_hbm.at[idx_c], rowbuf, sem_r)
            co = pltpu.async_copy(offs_hbm.at[idx_c], offbuf, sem_o)
            cr.wait()
            co.wait()
            pltpu.sync_copy(rowbuf, rows_out.at[pl.ds(base + off, CH)])
            pltpu.sync_copy(offbuf, offs_out.at[pl.ds(base + off, CH)])
            return carry

        lax.fori_loop(0, N_CH, body, 0)

    return _sc_gather


BLK = 1024
NBLK = E // BLK
NPOS = B_POS // BLK


def _tc_body(u_ref, v_ref, uo_ref, vo_ref, prob_ref, pos_ref, neg_ref):
    i = pl.program_id(0)
    u_min = u_ref[...]
    v_min = v_ref[...]
    uo = uo_ref[...]
    vo = vo_ref[...]
    sp = jax.nn.softplus
    u_max = u_min + sp(uo)
    v_max = v_min + sp(vo)
    inter_min = jnp.maximum(u_min, v_min)
    inter_max = jnp.minimum(u_max, v_max)
    inter_len = sp(inter_max - inter_min) + EPS
    v_len = sp(sp(vo)) + EPS
    logp = jnp.sum(jnp.log(inter_len), axis=1, keepdims=True) \
        - DIM * jnp.log(v_len)
    prob = jax.nn.sigmoid(logp)
    prob_ref[...] = prob
    lp = jnp.sum(jnp.log(prob + EPS))
    ln = jnp.sum(jnp.log(1.0 - prob + EPS))

    @pl.when(i == 0)
    def _():
        pos_ref[0, 0] = 0.0
        neg_ref[0, 0] = 0.0

    is_pos = i < NPOS
    pos_ref[0, 0] += jnp.where(is_pos, lp, 0.0)
    neg_ref[0, 0] += jnp.where(is_pos, 0.0, ln)


_tc_math = pl.pallas_call(
    _tc_body,
    grid=(NBLK,),
    in_specs=[
        pl.BlockSpec((BLK, DIM), lambda i: (i, 0)),
        pl.BlockSpec((BLK, DIM), lambda i: (i + NBLK, 0)),
        pl.BlockSpec((BLK, 1), lambda i: (i, 0)),
        pl.BlockSpec((BLK, 1), lambda i: (i + NBLK, 0)),
    ],
    out_specs=[
        pl.BlockSpec((BLK, 1), lambda i: (i, 0)),
        pl.BlockSpec(memory_space=pltpu.SMEM),
        pl.BlockSpec(memory_space=pltpu.SMEM),
    ],
    out_shape=[
        jax.ShapeDtypeStruct((E, 1), jnp.float32),
        jax.ShapeDtypeStruct((1, 1), jnp.float32),
        jax.ShapeDtypeStruct((1, 1), jnp.float32),
    ],
)


def kernel(embs, offset_embs, pos_u, pos_v, neg_u, neg_v):
    idx = jnp.concatenate([pos_u, neg_u, pos_v, neg_v]).astype(jnp.int32)
    offs_flat = offset_embs.reshape(N_NODES)
    rows, offs = _make_sc_gather()(embs, offs_flat, idx)
    offs2d = offs.reshape(TOT, 1)
    prob2d, ps, ns = _tc_math(rows, rows, offs2d, offs2d)
    loss = -(ps[0, 0] / B_POS) - (ns[0, 0] / B_NEG)
    edge_prob = prob2d.reshape(E)
    ground_truth = jnp.concatenate(
        [jnp.ones(B_POS, dtype=jnp.float32), jnp.zeros(B_NEG, dtype=jnp.float32)])
    return loss, edge_prob, ground_truth


# speedup vs baseline: 1.2979x; 1.2979x over previous
import functools
import math

import jax
import jax.numpy as jnp
from jax import lax
from jax.experimental import pallas as pl
from jax.experimental.pallas import tpu as pltpu
from jax.experimental.pallas import tpu_sc as plsc

N_NODES = 100000
DIM = 128
B_POS = 16384
B_NEG = 81920
EPS = 1e-7

E = B_POS + B_NEG
TOT = 2 * E
NW = 32
PER_W = TOT // NW
CH = 128
N_CH = PER_W // CH

LOG2E = 1.4426950408889634
LN2 = 0.6931471805599453
EPS2 = EPS / LN2
C128 = DIM * math.log(LN2)


@functools.cache
def _make_sc_gather():
    mesh = plsc.VectorSubcoreMesh(core_axis_name="c", subcore_axis_name="s")

    @functools.partial(
        pl.kernel,
        mesh=mesh,
        out_type=(
            jax.ShapeDtypeStruct((TOT, DIM), jnp.float32),
            jax.ShapeDtypeStruct((TOT,), jnp.float32),
        ),
        scratch_types=[
            pltpu.VMEM((PER_W,), jnp.int32),
            pltpu.VMEM((CH, DIM), jnp.float32),
            pltpu.VMEM((CH, DIM), jnp.float32),
            pltpu.VMEM((CH, DIM), jnp.float32),
            pltpu.VMEM((CH, DIM), jnp.float32),
            pltpu.VMEM((PER_W,), jnp.float32),
            pltpu.SemaphoreType.DMA,
            pltpu.SemaphoreType.DMA,
            pltpu.SemaphoreType.DMA,
            pltpu.SemaphoreType.DMA,
            pltpu.SemaphoreType.DMA,
            pltpu.SemaphoreType.DMA,
            pltpu.SemaphoreType.DMA,
            pltpu.SemaphoreType.DMA,
            pltpu.SemaphoreType.DMA,
        ],
    )
    def _sc_gather(embs_hbm, offs_hbm, idx_hbm, rows_out, offs_out,
                   idx_v, rb0, rb1, rb2, rb3, ob,
                   sg0, sg1, sg2, sg3, sw0, sw1, sw2, sw3, so):
        rbufs = (rb0, rb1, rb2, rb3)
        sgs = (sg0, sg1, sg2, sg3)
        sws = (sw0, sw1, sw2, sw3)
        wid = lax.axis_index("s") * 2 + lax.axis_index("c")
        base = wid * PER_W
        pltpu.sync_copy(idx_hbm.at[pl.ds(base, PER_W)], idx_v)

        def idx_at(c):
            return idx_v.at[pl.ds(c * CH, CH)]

        def start_g(c, p):
            pltpu.async_copy(embs_hbm.at[idx_at(c)], rbufs[p], sgs[p])

        def wait_g(c, p):
            pltpu.make_async_copy(embs_hbm.at[idx_at(c)], rbufs[p],
                                  sgs[p]).wait()

        def out_at(c):
            return rows_out.at[pl.ds(base + c * CH, CH)]

        def start_w(c, p):
            pltpu.async_copy(rbufs[p], out_at(c), sws[p])

        def wait_w(c, p):
            pltpu.make_async_copy(rbufs[p], out_at(c), sws[p]).wait()

        def obody(j, carry):
            pltpu.async_copy(offs_hbm.at[idx_at(j)],
                             ob.at[pl.ds(j * CH, CH)], so)
            return carry

        lax.fori_loop(0, N_CH, obody, 0)

        start_g(0, 0)
        start_g(1, 1)

        def rbody(j4, carry):
            for k in range(4):
                j = j4 * 4 + k
                pf = (k + 2) % 4

                @pl.when(j + 2 < N_CH)
                def _():
                    @pl.when(j >= 2)
                    def _():
                        wait_w(j - 2, pf)
                    start_g(j + 2, pf)

                wait_g(j, k)
                start_w(j, k)
            return carry

        lax.fori_loop(0, N_CH // 4, rbody, 0)
        for c in range(N_CH - 4, N_CH):
            wait_w(c, c % 4)

        pltpu.make_async_copy(offs_hbm.at[pl.ds(0, PER_W)], ob, so).wait()
        pltpu.sync_copy(ob, offs_out.at[pl.ds(base, PER_W)])

    return _sc_gather


ROWS_TOT = TOT // 128
ROWS_E = E // 128
ROWS_POS = B_POS // 128


def _off_body(o_ref, sp_ref, lvl_ref):
    x = o_ref[...]
    sp = jax.nn.softplus(x)
    sp_ref[...] = sp
    lvl_ref[...] = jnp.log(jax.nn.softplus(sp[ROWS_E:, :]) + EPS)


_off_kernel = pl.pallas_call(
    _off_body,
    out_shape=[
        jax.ShapeDtypeStruct((ROWS_TOT, 128), jnp.float32),
        jax.ShapeDtypeStruct((ROWS_E, 128), jnp.float32),
    ],
)


BLK = 1024
NBLK = E // BLK


def _sums_body(u_ref, v_ref, su_ref, sv_ref, sum_ref):
    u = u_ref[...]
    v = v_ref[...]
    su = su_ref[...]
    sv = sv_ref[...]
    d = jnp.minimum(u + su, v + sv) - jnp.maximum(u, v)
    t = jnp.exp2(d * LOG2E)
    s = jnp.log2(jnp.log2(1.0 + t) + EPS2)
    sum_ref[...] = jnp.sum(s, axis=1, keepdims=True)


_sums_kernel = pl.pallas_call(
    _sums_body,
    grid=(NBLK,),
    in_specs=[
        pl.BlockSpec((BLK, DIM), lambda i: (i, 0)),
        pl.BlockSpec((BLK, DIM), lambda i: (i + NBLK, 0)),
        pl.BlockSpec((BLK, 1), lambda i: (i, 0)),
        pl.BlockSpec((BLK, 1), lambda i: (i + NBLK, 0)),
    ],
    out_specs=[pl.BlockSpec((BLK, 1), lambda i: (i, 0))],
    out_shape=[jax.ShapeDtypeStruct((E, 1), jnp.float32)],
)


def _epi_body(s_ref, l_ref, p_ref, pos_ref, neg_ref):
    sums = s_ref[...]
    lvl = l_ref[...]
    logp = LN2 * sums + C128 - DIM * lvl
    prob = jax.nn.sigmoid(logp)
    p_ref[...] = prob
    rows = lax.broadcasted_iota(jnp.int32, (ROWS_E, 128), 0)
    is_pos = rows < ROWS_POS
    pos_ref[0, 0] = jnp.sum(jnp.where(is_pos, jnp.log(prob + EPS), 0.0))
    neg_ref[0, 0] = jnp.sum(jnp.where(is_pos, 0.0, jnp.log(1.0 - prob + EPS)))


_epi_kernel = pl.pallas_call(
    _epi_body,
    out_specs=[
        pl.BlockSpec((ROWS_E, 128), lambda: (0, 0)),
        pl.BlockSpec(memory_space=pltpu.SMEM),
        pl.BlockSpec(memory_space=pltpu.SMEM),
    ],
    out_shape=[
        jax.ShapeDtypeStruct((ROWS_E, 128), jnp.float32),
        jax.ShapeDtypeStruct((1, 1), jnp.float32),
        jax.ShapeDtypeStruct((1, 1), jnp.float32),
    ],
)


def kernel(embs, offset_embs, pos_u, pos_v, neg_u, neg_v):
    idx = jnp.concatenate([pos_u, neg_u, pos_v, neg_v]).astype(jnp.int32)
    offs_flat = offset_embs.reshape(N_NODES)
    rows, offs = _make_sc_gather()(embs, offs_flat, idx)
    sp, lvl = _off_kernel(offs.reshape(ROWS_TOT, 128))
    sp_col = sp.reshape(TOT, 1)
    (sums,) = _sums_kernel(rows, rows, sp_col, sp_col)
    prob, ps, ns = _epi_kernel(sums.reshape(ROWS_E, 128), lvl)
    loss = -(ps[0, 0] / B_POS) - (ns[0, 0] / B_NEG)
    edge_prob = prob.reshape(E)
    ground_truth = jnp.concatenate(
        [jnp.ones(B_POS, dtype=jnp.float32), jnp.zeros(B_NEG, dtype=jnp.float32)])
    return loss, edge_prob, ground_truth


# speedup vs baseline: 1.7688x; 1.3628x over previous
import functools
import math

import jax
import jax.numpy as jnp
from jax import lax
from jax.experimental import pallas as pl
from jax.experimental.pallas import tpu as pltpu
from jax.experimental.pallas import tpu_sc as plsc

N_NODES = 100000
DIM = 128
B_POS = 16384
B_NEG = 81920
EPS = 1e-7

E = B_POS + B_NEG
TOT = 2 * E
NW = 32
PER_W = TOT // NW
CH = 128
N_CH = PER_W // CH

LOG2E = 1.4426950408889634
LN2 = 0.6931471805599453
EPS2 = EPS / LN2
C128 = DIM * math.log(LN2)


@functools.cache
def _make_sc_gather():
    mesh = plsc.VectorSubcoreMesh(core_axis_name="c", subcore_axis_name="s")

    @functools.partial(
        pl.kernel,
        mesh=mesh,
        out_type=(
            jax.ShapeDtypeStruct((TOT, DIM), jnp.float32),
            jax.ShapeDtypeStruct((TOT,), jnp.float32),
        ),
        scratch_types=[
            pltpu.VMEM((PER_W,), jnp.int32),
            pltpu.VMEM((CH, DIM), jnp.float32),
            pltpu.VMEM((CH, DIM), jnp.float32),
            pltpu.VMEM((CH, DIM), jnp.float32),
            pltpu.VMEM((CH, DIM), jnp.float32),
            pltpu.VMEM((PER_W,), jnp.float32),
            pltpu.SemaphoreType.DMA,
            pltpu.SemaphoreType.DMA,
            pltpu.SemaphoreType.DMA,
            pltpu.SemaphoreType.DMA,
            pltpu.SemaphoreType.DMA,
            pltpu.SemaphoreType.DMA,
            pltpu.SemaphoreType.DMA,
            pltpu.SemaphoreType.DMA,
            pltpu.SemaphoreType.DMA,
        ],
    )
    def _sc_gather(embs_hbm, offs_hbm, idx_hbm, rows_out, offs_out,
                   idx_v, rb0, rb1, rb2, rb3, ob,
                   sg0, sg1, sg2, sg3, sw0, sw1, sw2, sw3, so):
        rbufs = (rb0, rb1, rb2, rb3)
        sgs = (sg0, sg1, sg2, sg3)
        sws = (sw0, sw1, sw2, sw3)
        wid = lax.axis_index("s") * 2 + lax.axis_index("c")
        base = wid * PER_W
        pltpu.sync_copy(idx_hbm.at[pl.ds(base, PER_W)], idx_v)

        def idx_at(c):
            return idx_v.at[pl.ds(c * CH, CH)]

        def start_g(c, p):
            pltpu.async_copy(embs_hbm.at[idx_at(c)], rbufs[p], sgs[p])

        def wait_g(c, p):
            pltpu.make_async_copy(embs_hbm.at[idx_at(c)], rbufs[p],
                                  sgs[p]).wait()

        def out_at(c):
            return rows_out.at[pl.ds(base + c * CH, CH)]

        def start_w(c, p):
            pltpu.async_copy(rbufs[p], out_at(c), sws[p])

        def wait_w(c, p):
            pltpu.make_async_copy(rbufs[p], out_at(c), sws[p]).wait()

        def obody(j, carry):
            pltpu.async_copy(offs_hbm.at[idx_at(j)],
                             ob.at[pl.ds(j * CH, CH)], so)
            return carry

        lax.fori_loop(0, N_CH, obody, 0)

        start_g(0, 0)
        start_g(1, 1)

        def rbody(j4, carry):
            for k in range(4):
                j = j4 * 4 + k
                pf = (k + 2) % 4

                @pl.when(j + 2 < N_CH)
                def _():
                    @pl.when(j >= 2)
                    def _():
                        wait_w(j - 2, pf)
                    start_g(j + 2, pf)

                wait_g(j, k)
                start_w(j, k)
            return carry

        lax.fori_loop(0, N_CH // 4, rbody, 0)
        for c in range(N_CH - 4, N_CH):
            wait_w(c, c % 4)

        pltpu.make_async_copy(offs_hbm.at[pl.ds(0, PER_W)], ob, so).wait()
        pltpu.sync_copy(ob, offs_out.at[pl.ds(base, PER_W)])

    return _sc_gather


ROWS_TOT = TOT // 128
ROWS_E = E // 128
ROWS_POS = B_POS // 128


def _off_body(o_ref, sp_ref, lvl_ref):
    x = o_ref[...]
    sp = jax.nn.softplus(x)
    sp_ref[...] = sp
    lvl_ref[...] = jnp.log(jax.nn.softplus(sp[ROWS_E:, :]) + EPS)


_off_kernel = pl.pallas_call(
    _off_body,
    out_shape=[
        jax.ShapeDtypeStruct((ROWS_TOT, 128), jnp.float32),
        jax.ShapeDtypeStruct((ROWS_E, 128), jnp.float32),
    ],
)


BLK = 1024
NBLK = E // BLK


BRK = BLK // 128


def _lane_to_sublane(lp):
    t = lp.T
    return jnp.concatenate([t[:, r:r + 1] for r in range(BRK)], axis=0)


def _sums_body(u_ref, v_ref, su_ref, sv_ref, sum_ref):
    u = u_ref[...]
    v = v_ref[...]
    su = _lane_to_sublane(su_ref[...])
    sv = _lane_to_sublane(sv_ref[...])
    d = jnp.minimum(u + su, v + sv) - jnp.maximum(u, v)
    t = jnp.exp2(d * LOG2E)
    s = jnp.log2(jnp.log2(1.0 + t) + EPS2)
    sum_ref[...] = jnp.sum(s, axis=1).reshape(BRK, 128)


_sums_kernel = pl.pallas_call(
    _sums_body,
    grid=(NBLK,),
    in_specs=[
        pl.BlockSpec((BLK, DIM), lambda i: (i, 0)),
        pl.BlockSpec((BLK, DIM), lambda i: (i + NBLK, 0)),
        pl.BlockSpec((BRK, 128), lambda i: (i, 0)),
        pl.BlockSpec((BRK, 128), lambda i: (i + NBLK, 0)),
    ],
    out_specs=[pl.BlockSpec((BRK, 128), lambda i: (i, 0))],
    out_shape=[jax.ShapeDtypeStruct((ROWS_E, 128), jnp.float32)],
)


def _epi_body(s_ref, l_ref, p_ref, pos_ref, neg_ref):
    sums = s_ref[...]
    lvl = l_ref[...]
    logp = LN2 * sums + C128 - DIM * lvl
    prob = jax.nn.sigmoid(logp)
    p_ref[...] = prob
    rows = lax.broadcasted_iota(jnp.int32, (ROWS_E, 128), 0)
    is_pos = rows < ROWS_POS
    pos_ref[0, 0] = jnp.sum(jnp.where(is_pos, jnp.log(prob + EPS), 0.0))
    neg_ref[0, 0] = jnp.sum(jnp.where(is_pos, 0.0, jnp.log(1.0 - prob + EPS)))


_epi_kernel = pl.pallas_call(
    _epi_body,
    out_specs=[
        pl.BlockSpec((ROWS_E, 128), lambda: (0, 0)),
        pl.BlockSpec(memory_space=pltpu.SMEM),
        pl.BlockSpec(memory_space=pltpu.SMEM),
    ],
    out_shape=[
        jax.ShapeDtypeStruct((ROWS_E, 128), jnp.float32),
        jax.ShapeDtypeStruct((1, 1), jnp.float32),
        jax.ShapeDtypeStruct((1, 1), jnp.float32),
    ],
)


def kernel(embs, offset_embs, pos_u, pos_v, neg_u, neg_v):
    idx = jnp.concatenate([pos_u, neg_u, pos_v, neg_v]).astype(jnp.int32)
    offs_flat = offset_embs.reshape(N_NODES)
    rows, offs = _make_sc_gather()(embs, offs_flat, idx)
    sp, lvl = _off_kernel(offs.reshape(ROWS_TOT, 128))
    (sums,) = _sums_kernel(rows, rows, sp, sp)
    prob, ps, ns = _epi_kernel(sums, lvl)
    loss = -(ps[0, 0] / B_POS) - (ns[0, 0] / B_NEG)
    edge_prob = prob.reshape(E)
    ground_truth = jnp.concatenate(
        [jnp.ones(B_POS, dtype=jnp.float32), jnp.zeros(B_NEG, dtype=jnp.float32)])
    return loss, edge_prob, ground_truth


# speedup vs baseline: 1.9980x; 1.1296x over previous
import functools
import math

import jax
import jax.numpy as jnp
from jax import lax
from jax.experimental import pallas as pl
from jax.experimental.pallas import tpu as pltpu
from jax.experimental.pallas import tpu_sc as plsc

N_NODES = 100000
DIM = 128
B_POS = 16384
B_NEG = 81920
EPS = 1e-7

E = B_POS + B_NEG
TOT = 2 * E
NW = 32
PER_W = TOT // NW
CH = 128
N_CH = PER_W // CH

LOG2E = 1.4426950408889634
LN2 = 0.6931471805599453
EPS2 = EPS / LN2
C128 = DIM * math.log(LN2)


@functools.cache
def _make_sc_gather():
    mesh = plsc.VectorSubcoreMesh(core_axis_name="c", subcore_axis_name="s")

    @functools.partial(
        pl.kernel,
        mesh=mesh,
        out_type=(
            jax.ShapeDtypeStruct((TOT, DIM), jnp.float32),
            jax.ShapeDtypeStruct((TOT,), jnp.float32),
        ),
        scratch_types=[
            pltpu.VMEM((PER_W,), jnp.int32),
            pltpu.VMEM((CH, DIM), jnp.float32),
            pltpu.VMEM((CH, DIM), jnp.float32),
            pltpu.VMEM((CH, DIM), jnp.float32),
            pltpu.VMEM((CH, DIM), jnp.float32),
            pltpu.VMEM((PER_W,), jnp.float32),
            pltpu.SemaphoreType.DMA,
            pltpu.SemaphoreType.DMA,
            pltpu.SemaphoreType.DMA,
            pltpu.SemaphoreType.DMA,
            pltpu.SemaphoreType.DMA,
            pltpu.SemaphoreType.DMA,
            pltpu.SemaphoreType.DMA,
            pltpu.SemaphoreType.DMA,
            pltpu.SemaphoreType.DMA,
        ],
    )
    def _sc_gather(embs_hbm, offs_hbm, idx_hbm, rows_out, offs_out,
                   idx_v, rb0, rb1, rb2, rb3, ob,
                   sg0, sg1, sg2, sg3, sw0, sw1, sw2, sw3, so):
        rbufs = (rb0, rb1, rb2, rb3)
        sgs = (sg0, sg1, sg2, sg3)
        sws = (sw0, sw1, sw2, sw3)
        wid = lax.axis_index("s") * 2 + lax.axis_index("c")
        base = wid * PER_W
        pltpu.sync_copy(idx_hbm.at[pl.ds(base, PER_W)], idx_v)

        def idx_at(c):
            return idx_v.at[pl.ds(c * CH, CH)]

        def start_g(c, p):
            pltpu.async_copy(embs_hbm.at[idx_at(c)], rbufs[p], sgs[p])

        def wait_g(c, p):
            pltpu.make_async_copy(embs_hbm.at[idx_at(c)], rbufs[p],
                                  sgs[p]).wait()

        def out_at(c):
            return rows_out.at[pl.ds(base + c * CH, CH)]

        def start_w(c, p):
            pltpu.async_copy(rbufs[p], out_at(c), sws[p])

        def wait_w(c, p):
            pltpu.make_async_copy(rbufs[p], out_at(c), sws[p]).wait()

        def obody(j, carry):
            pltpu.async_copy(offs_hbm.at[idx_at(j)],
                             ob.at[pl.ds(j * CH, CH)], so)
            return carry

        lax.fori_loop(0, N_CH, obody, 0)

        start_g(0, 0)
        start_g(1, 1)

        def rbody(j4, carry):
            for k in range(4):
                j = j4 * 4 + k
                pf = (k + 2) % 4

                @pl.when(j + 2 < N_CH)
                def _():
                    @pl.when(j >= 2)
                    def _():
                        wait_w(j - 2, pf)
                    start_g(j + 2, pf)

                wait_g(j, k)
                start_w(j, k)
            return carry

        lax.fori_loop(0, N_CH // 4, rbody, 0)
        for c in range(N_CH - 4, N_CH):
            wait_w(c, c % 4)

        pltpu.make_async_copy(offs_hbm.at[pl.ds(0, PER_W)], ob, so).wait()
        pltpu.sync_copy(ob, offs_out.at[pl.ds(base, PER_W)])

    return _sc_gather


ROWS_TOT = TOT // 128
ROWS_E = E // 128
ROWS_POS = B_POS // 128


def _off_body(o_ref, sp_ref, lvl_ref):
    x = o_ref[...]
    sp = jax.nn.softplus(x)
    sp_ref[...] = sp
    lvl_ref[...] = jnp.log(jax.nn.softplus(sp[ROWS_E:, :]) + EPS)


_off_kernel = pl.pallas_call(
    _off_body,
    out_shape=[
        jax.ShapeDtypeStruct((ROWS_TOT, 128), jnp.float32),
        jax.ShapeDtypeStruct((ROWS_E, 128), jnp.float32),
    ],
)


BLK = 1024
NBLK = E // BLK


BRK = BLK // 128


def _sums_body(u_ref, v_ref, su_ref, sv_ref, sum_ref):
    su = su_ref[...]
    sv = sv_ref[...]
    out_rows = []
    for r in range(BRK):
        ut = u_ref[r * 128:(r + 1) * 128, :].T
        vt = v_ref[r * 128:(r + 1) * 128, :].T
        d = jnp.minimum(ut + su[r:r + 1, :], vt + sv[r:r + 1, :]) \
            - jnp.maximum(ut, vt)
        t = jnp.exp2(d * LOG2E)
        s = jnp.log2(jnp.log2(1.0 + t) + EPS2)
        out_rows.append(jnp.sum(s, axis=0, keepdims=True))
    sum_ref[...] = jnp.concatenate(out_rows, axis=0)


_sums_kernel = pl.pallas_call(
    _sums_body,
    grid=(NBLK,),
    in_specs=[
        pl.BlockSpec((BLK, DIM), lambda i: (i, 0)),
        pl.BlockSpec((BLK, DIM), lambda i: (i + NBLK, 0)),
        pl.BlockSpec((BRK, 128), lambda i: (i, 0)),
        pl.BlockSpec((BRK, 128), lambda i: (i + NBLK, 0)),
    ],
    out_specs=[pl.BlockSpec((BRK, 128), lambda i: (i, 0))],
    out_shape=[jax.ShapeDtypeStruct((ROWS_E, 128), jnp.float32)],
)


def _epi_body(s_ref, l_ref, p_ref, pos_ref, neg_ref):
    sums = s_ref[...]
    lvl = l_ref[...]
    logp = LN2 * sums + C128 - DIM * lvl
    prob = jax.nn.sigmoid(logp)
    p_ref[...] = prob
    rows = lax.broadcasted_iota(jnp.int32, (ROWS_E, 128), 0)
    is_pos = rows < ROWS_POS
    pos_ref[0, 0] = jnp.sum(jnp.where(is_pos, jnp.log(prob + EPS), 0.0))
    neg_ref[0, 0] = jnp.sum(jnp.where(is_pos, 0.0, jnp.log(1.0 - prob + EPS)))


_epi_kernel = pl.pallas_call(
    _epi_body,
    out_specs=[
        pl.BlockSpec((ROWS_E, 128), lambda: (0, 0)),
        pl.BlockSpec(memory_space=pltpu.SMEM),
        pl.BlockSpec(memory_space=pltpu.SMEM),
    ],
    out_shape=[
        jax.ShapeDtypeStruct((ROWS_E, 128), jnp.float32),
        jax.ShapeDtypeStruct((1, 1), jnp.float32),
        jax.ShapeDtypeStruct((1, 1), jnp.float32),
    ],
)


def kernel(embs, offset_embs, pos_u, pos_v, neg_u, neg_v):
    idx = jnp.concatenate([pos_u, neg_u, pos_v, neg_v]).astype(jnp.int32)
    offs_flat = offset_embs.reshape(N_NODES)
    rows, offs = _make_sc_gather()(embs, offs_flat, idx)
    sp, lvl = _off_kernel(offs.reshape(ROWS_TOT, 128))
    (sums,) = _sums_kernel(rows, rows, sp, sp)
    prob, ps, ns = _epi_kernel(sums, lvl)
    loss = -(ps[0, 0] / B_POS) - (ns[0, 0] / B_NEG)
    edge_prob = prob.reshape(E)
    ground_truth = jnp.concatenate(
        [jnp.ones(B_POS, dtype=jnp.float32), jnp.zeros(B_NEG, dtype=jnp.float32)])
    return loss, edge_prob, ground_truth


# speedup vs baseline: 2.5950x; 1.2988x over previous
import functools
import math

import jax
import jax.numpy as jnp
from jax import lax
from jax.experimental import pallas as pl
from jax.experimental.pallas import tpu as pltpu
from jax.experimental.pallas import tpu_sc as plsc

N_NODES = 100000
DIM = 128
B_POS = 16384
B_NEG = 81920
EPS = 1e-7

E = B_POS + B_NEG
TOT = 2 * E
NW = 32
PER_W = TOT // NW
CH = 128
N_CH = PER_W // CH

LOG2E = 1.4426950408889634
LN2 = 0.6931471805599453
EPS2 = EPS / LN2
C128 = DIM * math.log(LN2)


@functools.cache
def _make_sc_gather():
    mesh = plsc.VectorSubcoreMesh(core_axis_name="c", subcore_axis_name="s")

    @functools.partial(
        pl.kernel,
        mesh=mesh,
        out_type=(
            jax.ShapeDtypeStruct((TOT, DIM), jnp.float32),
            jax.ShapeDtypeStruct((TOT,), jnp.float32),
        ),
        scratch_types=[
            pltpu.VMEM((PER_W,), jnp.int32),
            pltpu.VMEM((CH, DIM), jnp.float32),
            pltpu.VMEM((CH, DIM), jnp.float32),
            pltpu.VMEM((CH, DIM), jnp.float32),
            pltpu.VMEM((CH, DIM), jnp.float32),
            pltpu.VMEM((PER_W,), jnp.float32),
            pltpu.SemaphoreType.DMA,
            pltpu.SemaphoreType.DMA,
            pltpu.SemaphoreType.DMA,
            pltpu.SemaphoreType.DMA,
            pltpu.SemaphoreType.DMA,
            pltpu.SemaphoreType.DMA,
            pltpu.SemaphoreType.DMA,
            pltpu.SemaphoreType.DMA,
            pltpu.SemaphoreType.DMA,
        ],
    )
    def _sc_gather(embs_hbm, offs_hbm, idx_hbm, rows_out, offs_out,
                   idx_v, rb0, rb1, rb2, rb3, ob,
                   sg0, sg1, sg2, sg3, sw0, sw1, sw2, sw3, so):
        rbufs = (rb0, rb1, rb2, rb3)
        sgs = (sg0, sg1, sg2, sg3)
        sws = (sw0, sw1, sw2, sw3)
        wid = lax.axis_index("s") * 2 + lax.axis_index("c")
        base = wid * PER_W
        pltpu.sync_copy(idx_hbm.at[pl.ds(base, PER_W)], idx_v)

        def idx_at(c):
            return idx_v.at[pl.ds(c * CH, CH)]

        def start_g(c, p):
            pltpu.async_copy(embs_hbm.at[idx_at(c)], rbufs[p], sgs[p])

        def wait_g(c, p):
            pltpu.make_async_copy(embs_hbm.at[idx_at(c)], rbufs[p],
                                  sgs[p]).wait()

        def out_at(c):
            return rows_out.at[pl.ds(base + c * CH, CH)]

        def start_w(c, p):
            pltpu.async_copy(rbufs[p], out_at(c), sws[p])

        def wait_w(c, p):
            pltpu.make_async_copy(rbufs[p], out_at(c), sws[p]).wait()

        def obody(j, carry):
            pltpu.async_copy(offs_hbm.at[idx_at(j)],
                             ob.at[pl.ds(j * CH, CH)], so)
            return carry

        lax.fori_loop(0, N_CH, obody, 0)

        start_g(0, 0)
        start_g(1, 1)

        def rbody(j4, carry):
            for k in range(4):
                j = j4 * 4 + k
                pf = (k + 2) % 4

                @pl.when(j + 2 < N_CH)
                def _():
                    @pl.when(j >= 2)
                    def _():
                        wait_w(j - 2, pf)
                    start_g(j + 2, pf)

                wait_g(j, k)
                start_w(j, k)
            return carry

        lax.fori_loop(0, N_CH // 4, rbody, 0)
        for c in range(N_CH - 4, N_CH):
            wait_w(c, c % 4)

        pltpu.make_async_copy(offs_hbm.at[pl.ds(0, PER_W)], ob, so).wait()
        pltpu.sync_copy(ob, offs_out.at[pl.ds(base, PER_W)])

    return _sc_gather


ROWS_TOT = TOT // 128
ROWS_E = E // 128
ROWS_POS = B_POS // 128


def _off_body(o_ref, sp_ref, lvl_ref):
    x = o_ref[...]
    sp = jax.nn.softplus(x)
    sp_ref[...] = sp
    lvl_ref[...] = jnp.log(jax.nn.softplus(sp[ROWS_E:, :]) + EPS)


_off_kernel = pl.pallas_call(
    _off_body,
    out_shape=[
        jax.ShapeDtypeStruct((ROWS_TOT, 128), jnp.float32),
        jax.ShapeDtypeStruct((ROWS_E, 128), jnp.float32),
    ],
)


BLK = 8192
NBLK = E // BLK


BRK = BLK // 128


def _sums_body(u_ref, v_ref, su_ref, sv_ref, sum_ref):
    su = su_ref[...]
    sv = sv_ref[...]
    out_rows = []
    for r in range(BRK):
        ut = u_ref[r * 128:(r + 1) * 128, :].T
        vt = v_ref[r * 128:(r + 1) * 128, :].T
        d = jnp.minimum(ut + su[r:r + 1, :], vt + sv[r:r + 1, :]) \
            - jnp.maximum(ut, vt)
        t = jnp.exp2(d * LOG2E)
        s = jnp.log2(jnp.log2(1.0 + t) + EPS2)
        out_rows.append(jnp.sum(s, axis=0, keepdims=True))
    sum_ref[...] = jnp.concatenate(out_rows, axis=0)


_sums_kernel = pl.pallas_call(
    _sums_body,
    grid=(NBLK,),
    in_specs=[
        pl.BlockSpec((BLK, DIM), lambda i: (i, 0)),
        pl.BlockSpec((BLK, DIM), lambda i: (i + NBLK, 0)),
        pl.BlockSpec((BRK, 128), lambda i: (i, 0)),
        pl.BlockSpec((BRK, 128), lambda i: (i + NBLK, 0)),
    ],
    out_specs=[pl.BlockSpec((BRK, 128), lambda i: (i, 0))],
    out_shape=[jax.ShapeDtypeStruct((ROWS_E, 128), jnp.float32)],
)


def _epi_body(s_ref, l_ref, p_ref, pos_ref, neg_ref):
    sums = s_ref[...]
    lvl = l_ref[...]
    logp = LN2 * sums + C128 - DIM * lvl
    prob = jax.nn.sigmoid(logp)
    p_ref[...] = prob
    rows = lax.broadcasted_iota(jnp.int32, (ROWS_E, 128), 0)
    is_pos = rows < ROWS_POS
    pos_ref[0, 0] = jnp.sum(jnp.where(is_pos, jnp.log(prob + EPS), 0.0))
    neg_ref[0, 0] = jnp.sum(jnp.where(is_pos, 0.0, jnp.log(1.0 - prob + EPS)))


_epi_kernel = pl.pallas_call(
    _epi_body,
    out_specs=[
        pl.BlockSpec((ROWS_E, 128), lambda: (0, 0)),
        pl.BlockSpec(memory_space=pltpu.SMEM),
        pl.BlockSpec(memory_space=pltpu.SMEM),
    ],
    out_shape=[
        jax.ShapeDtypeStruct((ROWS_E, 128), jnp.float32),
        jax.ShapeDtypeStruct((1, 1), jnp.float32),
        jax.ShapeDtypeStruct((1, 1), jnp.float32),
    ],
)


def kernel(embs, offset_embs, pos_u, pos_v, neg_u, neg_v):
    idx = jnp.concatenate([pos_u, neg_u, pos_v, neg_v]).astype(jnp.int32)
    offs_flat = offset_embs.reshape(N_NODES)
    rows, offs = _make_sc_gather()(embs, offs_flat, idx)
    sp, lvl = _off_kernel(offs.reshape(ROWS_TOT, 128))
    (sums,) = _sums_kernel(rows, rows, sp, sp)
    prob, ps, ns = _epi_kernel(sums, lvl)
    loss = -(ps[0, 0] / B_POS) - (ns[0, 0] / B_NEG)
    edge_prob = prob.reshape(E)
    ground_truth = jnp.concatenate(
        [jnp.ones(B_POS, dtype=jnp.float32), jnp.zeros(B_NEG, dtype=jnp.float32)])
    return loss, edge_prob, ground_truth


# speedup vs baseline: 2.6556x; 1.0233x over previous
import functools
import math

import jax
import jax.numpy as jnp
from jax import lax
from jax.experimental import pallas as pl
from jax.experimental.pallas import tpu as pltpu
from jax.experimental.pallas import tpu_sc as plsc

N_NODES = 100000
DIM = 128
B_POS = 16384
B_NEG = 81920
EPS = 1e-7

E = B_POS + B_NEG
TOT = 2 * E
NW = 32
PER_W = TOT // NW
CH = 128
N_CH = PER_W // CH

LOG2E = 1.4426950408889634
LN2 = 0.6931471805599453
EPS2 = EPS / LN2
C128 = DIM * math.log(LN2)


@functools.cache
def _make_sc_gather():
    mesh = plsc.VectorSubcoreMesh(core_axis_name="c", subcore_axis_name="s")

    @functools.partial(
        pl.kernel,
        mesh=mesh,
        out_type=(
            jax.ShapeDtypeStruct((TOT, DIM), jnp.float32),
            jax.ShapeDtypeStruct((TOT,), jnp.float32),
        ),
        scratch_types=[
            pltpu.VMEM((PER_W,), jnp.int32),
            pltpu.VMEM((CH, DIM), jnp.float32),
            pltpu.VMEM((CH, DIM), jnp.float32),
            pltpu.VMEM((CH, DIM), jnp.float32),
            pltpu.VMEM((CH, DIM), jnp.float32),
            pltpu.VMEM((PER_W,), jnp.float32),
            pltpu.SemaphoreType.DMA,
            pltpu.SemaphoreType.DMA,
            pltpu.SemaphoreType.DMA,
            pltpu.SemaphoreType.DMA,
            pltpu.SemaphoreType.DMA,
            pltpu.SemaphoreType.DMA,
            pltpu.SemaphoreType.DMA,
            pltpu.SemaphoreType.DMA,
            pltpu.SemaphoreType.DMA,
        ],
    )
    def _sc_gather(embs_hbm, offs_hbm, idx_hbm, rows_out, offs_out,
                   idx_v, rb0, rb1, rb2, rb3, ob,
                   sg0, sg1, sg2, sg3, sw0, sw1, sw2, sw3, so):
        rbufs = (rb0, rb1, rb2, rb3)
        sgs = (sg0, sg1, sg2, sg3)
        sws = (sw0, sw1, sw2, sw3)
        wid = lax.axis_index("s") * 2 + lax.axis_index("c")
        base = wid * PER_W
        pltpu.sync_copy(idx_hbm.at[pl.ds(base, PER_W)], idx_v)

        def idx_at(c):
            return idx_v.at[pl.ds(c * CH, CH)]

        def start_g(c, p):
            pltpu.async_copy(embs_hbm.at[idx_at(c)], rbufs[p], sgs[p])

        def wait_g(c, p):
            pltpu.make_async_copy(embs_hbm.at[idx_at(c)], rbufs[p],
                                  sgs[p]).wait()

        def out_at(c):
            return rows_out.at[pl.ds(base + c * CH, CH)]

        def start_w(c, p):
            pltpu.async_copy(rbufs[p], out_at(c), sws[p])

        def wait_w(c, p):
            pltpu.make_async_copy(rbufs[p], out_at(c), sws[p]).wait()

        def obody(j, carry):
            pltpu.async_copy(offs_hbm.at[idx_at(j)],
                             ob.at[pl.ds(j * CH, CH)], so)
            return carry

        lax.fori_loop(0, N_CH, obody, 0)

        start_g(0, 0)
        start_g(1, 1)

        def rbody(j4, carry):
            for k in range(4):
                j = j4 * 4 + k
                pf = (k + 2) % 4

                @pl.when(j + 2 < N_CH)
                def _():
                    @pl.when(j >= 2)
                    def _():
                        wait_w(j - 2, pf)
                    start_g(j + 2, pf)

                wait_g(j, k)
                start_w(j, k)
            return carry

        lax.fori_loop(0, N_CH // 4, rbody, 0)
        for c in range(N_CH - 4, N_CH):
            wait_w(c, c % 4)

        pltpu.make_async_copy(offs_hbm.at[pl.ds(0, PER_W)], ob, so).wait()
        pltpu.sync_copy(ob, offs_out.at[pl.ds(base, PER_W)])

    return _sc_gather


ROWS_TOT = TOT // 128
ROWS_E = E // 128

BLK = 8192
NBLK = E // BLK
NPOS = B_POS // BLK
BRK = BLK // 128


def _softplus2(x):
    return jnp.log2(1.0 + jnp.exp2(x * LOG2E))


def _math_body(u_ref, v_ref, ou_ref, ov_ref, prob_ref, pos_ref, neg_ref):
    i = pl.program_id(0)
    su = LN2 * _softplus2(ou_ref[...])
    sv = LN2 * _softplus2(ov_ref[...])
    out_rows = []
    for r in range(BRK):
        ut = u_ref[r * 128:(r + 1) * 128, :].T
        vt = v_ref[r * 128:(r + 1) * 128, :].T
        d = jnp.minimum(ut + su[r:r + 1, :], vt + sv[r:r + 1, :]) \
            - jnp.maximum(ut, vt)
        t = jnp.exp2(d * LOG2E)
        s = jnp.log2(jnp.log2(1.0 + t) + EPS2)
        out_rows.append(jnp.sum(s, axis=0, keepdims=True))
    sums = jnp.concatenate(out_rows, axis=0)
    lvl2 = jnp.log2(_softplus2(sv) + EPS2)
    logp = LN2 * (sums - DIM * lvl2)
    prob = jax.nn.sigmoid(logp)
    prob_ref[...] = prob
    lp = jnp.sum(jnp.log(prob + EPS))
    ln = jnp.sum(jnp.log(1.0 - prob + EPS))

    @pl.when(i == 0)
    def _():
        pos_ref[0, 0] = 0.0
        neg_ref[0, 0] = 0.0

    is_pos = i < NPOS
    pos_ref[0, 0] += jnp.where(is_pos, lp, 0.0)
    neg_ref[0, 0] += jnp.where(is_pos, 0.0, ln)


_math_kernel = pl.pallas_call(
    _math_body,
    grid=(NBLK,),
    in_specs=[
        pl.BlockSpec((BLK, DIM), lambda i: (i, 0)),
        pl.BlockSpec((BLK, DIM), lambda i: (i + NBLK, 0)),
        pl.BlockSpec((BRK, 128), lambda i: (i, 0)),
        pl.BlockSpec((BRK, 128), lambda i: (i + NBLK, 0)),
    ],
    out_specs=[
        pl.BlockSpec((BRK, 128), lambda i: (i, 0)),
        pl.BlockSpec(memory_space=pltpu.SMEM),
        pl.BlockSpec(memory_space=pltpu.SMEM),
    ],
    out_shape=[
        jax.ShapeDtypeStruct((ROWS_E, 128), jnp.float32),
        jax.ShapeDtypeStruct((1, 1), jnp.float32),
        jax.ShapeDtypeStruct((1, 1), jnp.float32),
    ],
)


def kernel(embs, offset_embs, pos_u, pos_v, neg_u, neg_v):
    idx = jnp.concatenate([pos_u, neg_u, pos_v, neg_v]).astype(jnp.int32)
    offs_flat = offset_embs.reshape(N_NODES)
    rows, offs = _make_sc_gather()(embs, offs_flat, idx)
    offs_lane = offs.reshape(ROWS_TOT, 128)
    prob, ps, ns = _math_kernel(rows, rows, offs_lane, offs_lane)
    loss = -(ps[0, 0] / B_POS) - (ns[0, 0] / B_NEG)
    edge_prob = prob.reshape(E)
    ground_truth = jnp.concatenate(
        [jnp.ones(B_POS, dtype=jnp.float32), jnp.zeros(B_NEG, dtype=jnp.float32)])
    return loss, edge_prob, ground_truth


# speedup vs baseline: 2.6634x; 1.0029x over previous
import functools
import math

import jax
import jax.numpy as jnp
from jax import lax
from jax.experimental import pallas as pl
from jax.experimental.pallas import tpu as pltpu
from jax.experimental.pallas import tpu_sc as plsc

N_NODES = 100000
DIM = 128
B_POS = 16384
B_NEG = 81920
EPS = 1e-7

E = B_POS + B_NEG
TOT = 2 * E
NW = 32
PER_W = TOT // NW
CH = 128
N_CH = PER_W // CH

LOG2E = 1.4426950408889634
LN2 = 0.6931471805599453
EPS2 = EPS / LN2
C128 = DIM * math.log(LN2)


@functools.cache
def _make_sc_gather():
    mesh = plsc.VectorSubcoreMesh(core_axis_name="c", subcore_axis_name="s")

    NB = 6
    scratch_types = [pltpu.VMEM((PER_W,), jnp.int32)]
    scratch_types += [pltpu.VMEM((CH, DIM), jnp.float32)] * NB
    scratch_types += [pltpu.VMEM((PER_W,), jnp.float32)]
    scratch_types += [pltpu.SemaphoreType.DMA] * (2 * NB + 1)

    @functools.partial(
        pl.kernel,
        mesh=mesh,
        out_type=(
            jax.ShapeDtypeStruct((TOT, DIM), jnp.float32),
            jax.ShapeDtypeStruct((TOT,), jnp.float32),
        ),
        scratch_types=scratch_types,
    )
    def _sc_gather(embs_hbm, offs_hbm, idx_hbm, rows_out, offs_out,
                   idx_v, *scr):
        rbufs = scr[:NB]
        ob = scr[NB]
        sgs = scr[NB + 1:2 * NB + 1]
        sws = scr[2 * NB + 1:3 * NB + 1]
        so = scr[3 * NB + 1]
        wid = lax.axis_index("s") * 2 + lax.axis_index("c")
        base = wid * PER_W
        pltpu.sync_copy(idx_hbm.at[pl.ds(base, PER_W)], idx_v)

        def idx_at(c):
            return idx_v.at[pl.ds(c * CH, CH)]

        def start_g(c, p):
            pltpu.async_copy(embs_hbm.at[idx_at(c)], rbufs[p], sgs[p])

        def wait_g(c, p):
            pltpu.make_async_copy(embs_hbm.at[idx_at(c)], rbufs[p],
                                  sgs[p]).wait()

        def out_at(c):
            return rows_out.at[pl.ds(base + c * CH, CH)]

        def start_w(c, p):
            pltpu.async_copy(rbufs[p], out_at(c), sws[p])

        def wait_w(c, p):
            pltpu.make_async_copy(rbufs[p], out_at(c), sws[p]).wait()

        def obody(j, carry):
            pltpu.async_copy(offs_hbm.at[idx_at(j)],
                             ob.at[pl.ds(j * CH, CH)], so)
            return carry

        lax.fori_loop(0, N_CH, obody, 0)

        PF = NB // 2
        for c in range(PF):
            start_g(c, c)

        def rbody(jo, carry):
            for k in range(NB):
                j = jo * NB + k
                pf = (k + PF) % NB

                @pl.when(j + PF < N_CH)
                def _():
                    @pl.when(j >= PF)
                    def _():
                        wait_w(j - PF, pf)
                    start_g(j + PF, pf)

                wait_g(j, k)
                start_w(j, k)
            return carry

        lax.fori_loop(0, N_CH // NB, rbody, 0)
        for c in range(N_CH - 2 * PF, N_CH):
            wait_w(c, c % NB)

        pltpu.make_async_copy(offs_hbm.at[pl.ds(0, PER_W)], ob, so).wait()
        pltpu.sync_copy(ob, offs_out.at[pl.ds(base, PER_W)])

    return _sc_gather


ROWS_TOT = TOT // 128
ROWS_E = E // 128

BLK = 8192
NBLK = E // BLK
NPOS = B_POS // BLK
BRK = BLK // 128


def _softplus2(x):
    return jnp.log2(1.0 + jnp.exp2(x * LOG2E))


def _math_body(u_ref, v_ref, ou_ref, ov_ref, prob_ref, pos_ref, neg_ref):
    i = pl.program_id(0)
    su = LN2 * _softplus2(ou_ref[...])
    sv = LN2 * _softplus2(ov_ref[...])
    out_rows = []
    for r in range(BRK):
        ut = u_ref[r * 128:(r + 1) * 128, :].T
        vt = v_ref[r * 128:(r + 1) * 128, :].T
        d = jnp.minimum(ut + su[r:r + 1, :], vt + sv[r:r + 1, :]) \
            - jnp.maximum(ut, vt)
        t = jnp.exp2(d * LOG2E)
        s = jnp.log2(jnp.log2(1.0 + t) + EPS2)
        out_rows.append(jnp.sum(s, axis=0, keepdims=True))
    sums = jnp.concatenate(out_rows, axis=0)
    lvl2 = jnp.log2(_softplus2(sv) + EPS2)
    logp = LN2 * (sums - DIM * lvl2)
    prob = jax.nn.sigmoid(logp)
    prob_ref[...] = prob
    lp = jnp.sum(jnp.log(prob + EPS))
    ln = jnp.sum(jnp.log(1.0 - prob + EPS))

    @pl.when(i == 0)
    def _():
        pos_ref[0, 0] = 0.0
        neg_ref[0, 0] = 0.0

    is_pos = i < NPOS
    pos_ref[0, 0] += jnp.where(is_pos, lp, 0.0)
    neg_ref[0, 0] += jnp.where(is_pos, 0.0, ln)


_math_kernel = pl.pallas_call(
    _math_body,
    grid=(NBLK,),
    in_specs=[
        pl.BlockSpec((BLK, DIM), lambda i: (i, 0)),
        pl.BlockSpec((BLK, DIM), lambda i: (i + NBLK, 0)),
        pl.BlockSpec((BRK, 128), lambda i: (i, 0)),
        pl.BlockSpec((BRK, 128), lambda i: (i + NBLK, 0)),
    ],
    out_specs=[
        pl.BlockSpec((BRK, 128), lambda i: (i, 0)),
        pl.BlockSpec(memory_space=pltpu.SMEM),
        pl.BlockSpec(memory_space=pltpu.SMEM),
    ],
    out_shape=[
        jax.ShapeDtypeStruct((ROWS_E, 128), jnp.float32),
        jax.ShapeDtypeStruct((1, 1), jnp.float32),
        jax.ShapeDtypeStruct((1, 1), jnp.float32),
    ],
)


def kernel(embs, offset_embs, pos_u, pos_v, neg_u, neg_v):
    idx = jnp.concatenate([pos_u, neg_u, pos_v, neg_v]).astype(jnp.int32)
    offs_flat = offset_embs.reshape(N_NODES)
    rows, offs = _make_sc_gather()(embs, offs_flat, idx)
    offs_lane = offs.reshape(ROWS_TOT, 128)
    prob, ps, ns = _math_kernel(rows, rows, offs_lane, offs_lane)
    loss = -(ps[0, 0] / B_POS) - (ns[0, 0] / B_NEG)
    edge_prob = prob.reshape(E)
    ground_truth = jnp.concatenate(
        [jnp.ones(B_POS, dtype=jnp.float32), jnp.zeros(B_NEG, dtype=jnp.float32)])
    return loss, edge_prob, ground_truth
